# Initial kernel scaffold; baseline (speedup 1.0000x reference)
import functools

import jax
import jax.numpy as jnp
from jax import lax
from jax.experimental import pallas as pl
from jax.experimental.pallas import tpu as pltpu
from jax.experimental.pallas import tpu_sc as plsc

N = 10000
E = 320000
D = 128

NC = 2
NS = 16
NWORK = NC * NS
EPW = E // NWORK
K = 400
KR = 5
KC = K // KR
NCHUNK = EPW // K
IDX_ROWS_PER_WORKER = EPW // KC
DEGW = 16
NZCH = (N + K - 1) // K


def _sc_agg_body(with_deg, x_hbm, src_hbm, dst_hbm, ew_hbm, *refs):
    if with_deg:
        (s0_hbm, s1_hbm, d0_hbm, d1_hbm,
         src_v, dst_v, ew_v, rows_v, w16_v, acc_s, deg_s, sem) = refs
    else:
        (s0_hbm, s1_hbm,
         src_v, dst_v, ew_v, rows_v, w16_v, acc_s, deg_s, sem) = refs

    c = lax.axis_index("c")
    s = lax.axis_index("s")
    wid = s * NC + c

    def zb(i, carry):
        for j in range(D // 16):
            rows_v[i, pl.ds(j * 16, 16)] = jnp.zeros((16,), jnp.float32)
        w16_v[i, :] = jnp.zeros((16,), jnp.float32)
        return carry
    lax.fori_loop(0, K, zb, 0)

    def zc(i, carry):
        ci = i * NS + s
        @pl.when(ci < NZCH)
        def _():
            pltpu.sync_copy(rows_v, acc_s.at[pl.ds(ci * K, K)])
            if with_deg:
                pltpu.sync_copy(w16_v, deg_s.at[pl.ds(ci * K, K)])
        return carry
    lax.fori_loop(0, (NZCH + NS - 1) // NS, zc, 0)
    plsc.subcore_barrier()

    idx_base = wid * IDX_ROWS_PER_WORKER
    ew_base = wid * EPW

    def chunk_body(ci, carry):
        r0 = idx_base + ci * KR
        pltpu.sync_copy(src_hbm.at[pl.ds(r0, KR)], src_v)
        pltpu.sync_copy(dst_hbm.at[pl.ds(r0, KR)], dst_v)
        pltpu.sync_copy(ew_hbm.at[pl.ds(ew_base + ci * K, K)], ew_v)
        cps = [pltpu.async_copy(x_hbm.at[src_v.at[r]],
                                rows_v.at[pl.ds(r * KC, KC)], sem)
               for r in range(KR)]
        for cp in cps:
            cp.wait()

        def edge_body(e, carry2):
            wv = plsc.load_gather(ew_v, [jnp.full((16,), e, jnp.int32)])
            if with_deg:
                w16_v[e, :] = wv
            for j in range(D // 16):
                r = rows_v[e, pl.ds(j * 16, 16)]
                rows_v[e, pl.ds(j * 16, 16)] = r * wv
            return carry2
        lax.fori_loop(0, K, edge_body, 0)

        for r in range(KR):
            pltpu.sync_copy(rows_v.at[pl.ds(r * KC, KC)],
                            acc_s.at[dst_v.at[r]], add=True)
            if with_deg:
                pltpu.sync_copy(w16_v.at[pl.ds(r * KC, KC)],
                                deg_s.at[dst_v.at[r]], add=True)
        return carry
    lax.fori_loop(0, NCHUNK, chunk_body, 0)
    plsc.subcore_barrier()

    sbase = s * 624
    s_hbm = s0_hbm
    d_hbm = d0_hbm if with_deg else None

    def writeout(s_out, d_out):
        pltpu.sync_copy(acc_s.at[pl.ds(sbase, 400)],
                        s_out.at[pl.ds(sbase, 400)])
        pltpu.sync_copy(acc_s.at[pl.ds(sbase + 400, 224)],
                        s_out.at[pl.ds(sbase + 400, 224)])
        if with_deg:
            pltpu.sync_copy(deg_s.at[pl.ds(sbase, 400)],
                            d_out.at[pl.ds(sbase, 400)])
            pltpu.sync_copy(deg_s.at[pl.ds(sbase + 400, 224)],
                            d_out.at[pl.ds(sbase + 400, 224)])
        @pl.when(s == NS - 1)
        def _():
            pltpu.sync_copy(acc_s.at[pl.ds(9984, 16)],
                            s_out.at[pl.ds(9984, 16)])
            if with_deg:
                pltpu.sync_copy(deg_s.at[pl.ds(9984, 16)],
                                d_out.at[pl.ds(9984, 16)])

    @pl.when(c == 0)
    def _():
        writeout(s0_hbm, d0_hbm if with_deg else None)

    @pl.when(c == 1)
    def _():
        writeout(s1_hbm, d1_hbm if with_deg else None)


def _make_sc_agg(with_deg):
    mesh = plsc.VectorSubcoreMesh(core_axis_name="c", subcore_axis_name="s")
    out_type = [jax.ShapeDtypeStruct((N, D), jnp.float32),
                jax.ShapeDtypeStruct((N, D), jnp.float32)]
    if with_deg:
        out_type += [jax.ShapeDtypeStruct((N, DEGW), jnp.float32),
                     jax.ShapeDtypeStruct((N, DEGW), jnp.float32)]
    return pl.kernel(
        functools.partial(_sc_agg_body, with_deg),
        out_type=out_type,
        mesh=mesh,
        scratch_types=[
            pltpu.VMEM((KR, KC), jnp.int32),
            pltpu.VMEM((KR, KC), jnp.int32),
            pltpu.VMEM((K,), jnp.float32),
            pltpu.VMEM((K, D), jnp.float32),
            pltpu.VMEM((K, DEGW), jnp.float32),
            pltpu.VMEM_SHARED((N, D), jnp.float32),
            pltpu.VMEM_SHARED((N, DEGW), jnp.float32),
            pltpu.SemaphoreType.DMA,
        ],
        name="sc_gin_agg_deg" if with_deg else "sc_gin_agg",
    )


_sc_agg_deg = _make_sc_agg(True)
_sc_agg = _make_sc_agg(False)

BN = 2000


def _tc_layer_body(x_ref, s0_ref, s1_ref, d0_ref, d1_ref, w_ref, b_ref,
                   o_ref):
    deg = d0_ref[:, 0:1] + d1_ref[:, 0:1]
    safe = jnp.where(deg > 0, deg, 1.0)
    z = x_ref[...] + (s0_ref[...] + s1_ref[...]) / safe
    h = lax.dot_general(z, w_ref[...], (((1,), (1,)), ((), ())),
                        preferred_element_type=jnp.float32) + b_ref[...]
    o_ref[...] = jnp.maximum(h, 0.0)


def _tc_layer(x, s0, s1, d0, d1, w, b):
    return pl.pallas_call(
        _tc_layer_body,
        grid=(N // BN,),
        in_specs=[
            pl.BlockSpec((BN, D), lambda i: (i, 0)),
            pl.BlockSpec((BN, D), lambda i: (i, 0)),
            pl.BlockSpec((BN, D), lambda i: (i, 0)),
            pl.BlockSpec((BN, DEGW), lambda i: (i, 0)),
            pl.BlockSpec((BN, DEGW), lambda i: (i, 0)),
            pl.BlockSpec((D, D), lambda i: (0, 0)),
            pl.BlockSpec((1, D), lambda i: (0, 0)),
        ],
        out_specs=pl.BlockSpec((BN, D), lambda i: (i, 0)),
        out_shape=jax.ShapeDtypeStruct((N, D), jnp.float32),
    )(x, s0, s1, d0, d1, w, b)


def _tc_head_body(h_ref, s0_ref, s1_ref, d0_ref, d1_ref, nw_ref,
                  w2_ref, b2_ref, wd_ref, bd_ref, wc_ref, bc_ref,
                  o_ref, acc_ref):
    i = pl.program_id(0)

    @pl.when(i == 0)
    def _():
        acc_ref[...] = jnp.zeros_like(acc_ref)

    deg = d0_ref[:, 0:1] + d1_ref[:, 0:1]
    safe = jnp.where(deg > 0, deg, 1.0)
    z = h_ref[...] + (s0_ref[...] + s1_ref[...]) / safe
    h2 = lax.dot_general(z, w2_ref[...], (((1,), (1,)), ((), ())),
                         preferred_element_type=jnp.float32) + b2_ref[...]
    h2 = jnp.maximum(h2, 0.0)
    acc_ref[...] += lax.dot_general(nw_ref[...], h2,
                                    (((1,), (0,)), ((), ())),
                                    preferred_element_type=jnp.float32)

    @pl.when(i == pl.num_programs(0) - 1)
    def _():
        hg = acc_ref[...] * (1.0 / N)
        o1 = lax.dot_general(hg, wd_ref[...], (((1,), (1,)), ((), ())),
                             preferred_element_type=jnp.float32) + bd_ref[...]
        o1 = jnp.maximum(o1, 0.0)
        o_ref[...] = lax.dot_general(o1, wc_ref[...],
                                     (((1,), (1,)), ((), ())),
                                     preferred_element_type=jnp.float32) \
            + bc_ref[...]


def _tc_head(h, s0, s1, d0, d1, nw_row, w2, b2, wd, bd, wc, bc):
    nh = wd.shape[0]
    nc = wc.shape[0]
    return pl.pallas_call(
        _tc_head_body,
        grid=(N // BN,),
        in_specs=[
            pl.BlockSpec((BN, D), lambda i: (i, 0)),
            pl.BlockSpec((BN, D), lambda i: (i, 0)),
            pl.BlockSpec((BN, D), lambda i: (i, 0)),
            pl.BlockSpec((BN, DEGW), lambda i: (i, 0)),
            pl.BlockSpec((BN, DEGW), lambda i: (i, 0)),
            pl.BlockSpec((1, BN), lambda i: (0, i)),
            pl.BlockSpec((D, D), lambda i: (0, 0)),
            pl.BlockSpec((1, D), lambda i: (0, 0)),
            pl.BlockSpec((nh, D), lambda i: (0, 0)),
            pl.BlockSpec((1, nh), lambda i: (0, 0)),
            pl.BlockSpec((nc, nh), lambda i: (0, 0)),
            pl.BlockSpec((1, nc), lambda i: (0, 0)),
        ],
        out_specs=pl.BlockSpec((1, nc), lambda i: (0, 0)),
        out_shape=jax.ShapeDtypeStruct((1, nc), jnp.float32),
        scratch_shapes=[pltpu.VMEM((1, D), jnp.float32)],
    )(h, s0, s1, d0, d1, nw_row, w2, b2, wd, bd, wc, bc)


@jax.jit
def kernel(in_feat, edge_index, edge_weights, node_weights,
           W1, b1, W2, b2, Wd, bd, Wc, bc):
    src2d = edge_index[0].reshape(E // KC, KC)
    dst2d = edge_index[1].reshape(E // KC, KC)

    s0, s1, d0, d1 = _sc_agg_deg(in_feat, src2d, dst2d, edge_weights)
    h = _tc_layer(in_feat, s0, s1, d0, d1, W1, b1.reshape(1, -1))
    t0, t1 = _sc_agg(h, src2d, dst2d, edge_weights)
    out = _tc_head(h, t0, t1, d0, d1, node_weights.reshape(1, N),
                   W2, b2.reshape(1, -1), Wd, bd.reshape(1, -1),
                   Wc, bc.reshape(1, -1))
    return out


# speedup vs baseline: 1.7762x; 1.7762x over previous
import jax
import jax.numpy as jnp
from jax import lax
from jax.experimental import pallas as pl
from jax.experimental.pallas import tpu as pltpu
from jax.experimental.pallas import tpu_sc as plsc

N = 10000
E = 320000
D = 128

NC = 2
NS = 16
NWORK = NC * NS
EPW = E // NWORK
KC = 50
KR = 8
K = KR * KC
NCHUNK = EPW // K
IDXR = EPW // KC
DEGW = 16
NPASS = 4
PN = 2512
NTRASH = 16
NH = PN + NTRASH
NHD16 = NH // 16
ZSTRIPE = 152


def _sc_agg_body(x_hbm, src_hbm, dst_hbm, ew16_hbm,
                 s0_hbm, s1_hbm,
                 src_v, dst_v, dstl_v, w16_v, rows_v, acc_s, sem):
    c = lax.axis_index("c")
    s = lax.axis_index("s")
    wid = s * NC + c
    idx_base = wid * IDXR
    lane = lax.iota(jnp.int32, 16)

    for p in range(NPASS):
        def zb(i, carry):
            for j in range(D // 16):
                rows_v[i, pl.ds(j * 16, 16)] = jnp.zeros((16,),
                                                         jnp.float32)
            return carry
        lax.fori_loop(0, ZSTRIPE + 32, zb, 0)

        pltpu.sync_copy(rows_v.at[pl.ds(0, ZSTRIPE)],
                        acc_s.at[pl.ds(s * ZSTRIPE, ZSTRIPE)])
        @pl.when(s == NS - 1)
        def _():
            ztail = NH - NS * ZSTRIPE
            pltpu.sync_copy(rows_v.at[pl.ds(0, ztail)],
                            acc_s.at[pl.ds(NS * ZSTRIPE, ztail)])
        plsc.subcore_barrier()

        def chunk_body(ci, carry):
            r0 = idx_base + ci * KR
            pltpu.sync_copy(src_hbm.at[pl.ds(r0, KR)], src_v)
            pltpu.sync_copy(dst_hbm.at[pl.ds(r0, KR)], dst_v)
            pltpu.sync_copy(ew16_hbm.at[pl.ds(wid * EPW + ci * K, K)],
                            w16_v)

            for r in range(KR):
                for off in (0, 16, 32, 34):
                    dd = dst_v[r, pl.ds(off, 16)]
                    ll = dd - (p * PN)
                    bad = (ll < 0) | (ll >= PN)
                    dstl_v[r, pl.ds(off, 16)] = jnp.where(bad, PN + lane,
                                                          ll)

            cps = [pltpu.async_copy(x_hbm.at[src_v.at[r]],
                                    rows_v.at[pl.ds(r * KC, KC)], sem)
                   for r in range(KR)]
            for cp in cps:
                cp.wait()

            def edge_body(e, carry2):
                wv = w16_v[e, :]
                for q in range(D // 16):
                    rr = rows_v[e, pl.ds(q * 16, 16)]
                    rows_v[e, pl.ds(q * 16, 16)] = rr * wv
                return carry2
            lax.fori_loop(0, K, edge_body, 0)

            for r in range(KR):
                pltpu.sync_copy(rows_v.at[pl.ds(r * KC, KC)],
                                acc_s.at[dstl_v.at[r]], add=True)
            return carry
        lax.fori_loop(0, NCHUNK, chunk_body, 0)
        plsc.subcore_barrier()

        rows_p = PN if p < NPASS - 1 else N - (NPASS - 1) * PN
        obase = p * PN + s * ZSTRIPE
        tail = rows_p - NS * ZSTRIPE

        def writeout(s_out):
            pltpu.sync_copy(acc_s.at[pl.ds(s * ZSTRIPE, ZSTRIPE)],
                            s_out.at[pl.ds(obase, ZSTRIPE)])
            if tail > 0:
                @pl.when(s == NS - 1)
                def _():
                    pltpu.sync_copy(acc_s.at[pl.ds(NS * ZSTRIPE, tail)],
                                    s_out.at[pl.ds(p * PN + NS * ZSTRIPE,
                                                   tail)])

        @pl.when(c == 0)
        def _():
            writeout(s0_hbm)

        @pl.when(c == 1)
        def _():
            writeout(s1_hbm)

        plsc.subcore_barrier()


_sc_agg = pl.kernel(
    _sc_agg_body,
    out_type=[jax.ShapeDtypeStruct((N, D), jnp.float32),
              jax.ShapeDtypeStruct((N, D), jnp.float32)],
    mesh=plsc.VectorSubcoreMesh(core_axis_name="c", subcore_axis_name="s"),
    scratch_types=[
        pltpu.VMEM((KR, KC), jnp.int32),
        pltpu.VMEM((KR, KC), jnp.int32),
        pltpu.VMEM((KR, KC), jnp.int32),
        pltpu.VMEM((K, DEGW), jnp.float32),
        pltpu.VMEM((K, D), jnp.float32),
        pltpu.VMEM_SHARED((NH, D), jnp.float32),
        pltpu.SemaphoreType.DMA,
    ],
    name="sc_gin_agg",
)

BN = 2000


BQ = 8000


def _tc_bcast_body(w_ref, o_ref):
    o_ref[...] = jnp.broadcast_to(w_ref[...], (BQ, DEGW))


def _tc_bcast(ew_col):
    return pl.pallas_call(
        _tc_bcast_body,
        grid=(E // BQ,),
        in_specs=[pl.BlockSpec((BQ, 1), lambda i: (i, 0))],
        out_specs=pl.BlockSpec((BQ, DEGW), lambda i: (i, 0)),
        out_shape=jax.ShapeDtypeStruct((E, DEGW), jnp.float32),
    )(ew_col)


def _agg_block(s0, s1, deg):
    safe = jnp.where(deg > 0, deg, 1.0)
    return (s0 + s1) / safe


def _tc_layer_body(x_ref, s0_ref, s1_ref, deg_ref, w_ref, b_ref,
                   o_ref):
    z = x_ref[...] + _agg_block(s0_ref[...], s1_ref[...], deg_ref[...])
    h = lax.dot_general(z, w_ref[...], (((1,), (1,)), ((), ())),
                        preferred_element_type=jnp.float32) + b_ref[...]
    o_ref[...] = jnp.maximum(h, 0.0)


def _tc_layer(x, s0, s1, deg, w, b):
    return pl.pallas_call(
        _tc_layer_body,
        grid=(N // BN,),
        in_specs=[
            pl.BlockSpec((BN, D), lambda i: (i, 0)),
            pl.BlockSpec((BN, D), lambda i: (i, 0)),
            pl.BlockSpec((BN, D), lambda i: (i, 0)),
            pl.BlockSpec((BN, 1), lambda i: (i, 0)),
            pl.BlockSpec((D, D), lambda i: (0, 0)),
            pl.BlockSpec((1, D), lambda i: (0, 0)),
        ],
        out_specs=pl.BlockSpec((BN, D), lambda i: (i, 0)),
        out_shape=jax.ShapeDtypeStruct((N, D), jnp.float32),
    )(x, s0, s1, deg, w, b)


def _tc_head_body(h_ref, s0_ref, s1_ref, deg_ref, nw_ref,
                  w2_ref, b2_ref, wd_ref, bd_ref, wc_ref, bc_ref,
                  o_ref, acc_ref):
    i = pl.program_id(0)

    @pl.when(i == 0)
    def _():
        acc_ref[...] = jnp.zeros_like(acc_ref)

    z = h_ref[...] + _agg_block(s0_ref[...], s1_ref[...], deg_ref[...])
    h2 = lax.dot_general(z, w2_ref[...], (((1,), (1,)), ((), ())),
                         preferred_element_type=jnp.float32) + b2_ref[...]
    h2 = jnp.maximum(h2, 0.0)
    acc_ref[...] += lax.dot_general(nw_ref[...], h2,
                                    (((0,), (0,)), ((), ())),
                                    preferred_element_type=jnp.float32)

    @pl.when(i == pl.num_programs(0) - 1)
    def _():
        hg = acc_ref[...] * (1.0 / N)
        o1 = lax.dot_general(hg, wd_ref[...], (((1,), (1,)), ((), ())),
                             preferred_element_type=jnp.float32) + bd_ref[...]
        o1 = jnp.maximum(o1, 0.0)
        o_ref[...] = lax.dot_general(o1, wc_ref[...],
                                     (((1,), (1,)), ((), ())),
                                     preferred_element_type=jnp.float32) \
            + bc_ref[...]


def _tc_head(h, s0, s1, deg, nw_col, w2, b2, wd, bd, wc, bc):
    nh = wd.shape[0]
    nc = wc.shape[0]
    return pl.pallas_call(
        _tc_head_body,
        grid=(N // BN,),
        in_specs=[
            pl.BlockSpec((BN, D), lambda i: (i, 0)),
            pl.BlockSpec((BN, D), lambda i: (i, 0)),
            pl.BlockSpec((BN, D), lambda i: (i, 0)),
            pl.BlockSpec((BN, 1), lambda i: (i, 0)),
            pl.BlockSpec((BN, 1), lambda i: (i, 0)),
            pl.BlockSpec((D, D), lambda i: (0, 0)),
            pl.BlockSpec((1, D), lambda i: (0, 0)),
            pl.BlockSpec((nh, D), lambda i: (0, 0)),
            pl.BlockSpec((1, nh), lambda i: (0, 0)),
            pl.BlockSpec((nc, nh), lambda i: (0, 0)),
            pl.BlockSpec((1, nc), lambda i: (0, 0)),
        ],
        out_specs=pl.BlockSpec((1, nc), lambda i: (0, 0)),
        out_shape=jax.ShapeDtypeStruct((1, nc), jnp.float32),
        scratch_shapes=[pltpu.VMEM((1, D), jnp.float32)],
    )(h, s0, s1, deg, nw_col, w2, b2, wd, bd, wc, bc)


@jax.jit
def kernel(in_feat, edge_index, edge_weights, node_weights,
           W1, b1, W2, b2, Wd, bd, Wc, bc):
    src2d = edge_index[0].reshape(E // KC, KC)
    dst2d = edge_index[1].reshape(E // KC, KC)
    ew16 = _tc_bcast(edge_weights.reshape(E, 1))

    deg = jax.ops.segment_sum(edge_weights, edge_index[1],
                              num_segments=N).reshape(N, 1)
    s0, s1 = _sc_agg(in_feat, src2d, dst2d, ew16)
    h = _tc_layer(in_feat, s0, s1, deg, W1, b1.reshape(1, -1))
    t0, t1 = _sc_agg(h, src2d, dst2d, ew16)
    out = _tc_head(h, t0, t1, deg, node_weights.reshape(N, 1),
                   W2, b2.reshape(1, -1), Wd, bd.reshape(1, -1),
                   Wc, bc.reshape(1, -1))
    return out


# speedup vs baseline: 1.7828x; 1.0037x over previous
import jax
import jax.numpy as jnp
from jax import lax
from jax.experimental import pallas as pl
from jax.experimental.pallas import tpu as pltpu
from jax.experimental.pallas import tpu_sc as plsc

N = 10000
E = 320000
D = 128

NC = 2
NS = 16
NWORK = NC * NS
EPW = E // NWORK
KC = 50
KR = 8
K = KR * KC
NCHUNK = EPW // K
IDXR = EPW // KC
DEGW = 16
NPASS = 4
PN = 2512
NTRASH = 16
NH = PN + NTRASH
ZSTRIPE = 152


def _sc_agg_body(x_hbm, src_hbm, dst_hbm, ew16_hbm,
                 s0_hbm, s1_hbm,
                 src_v, dst_v, dstl_v, w16_v, rows_v, acc_s, sem):
    c = lax.axis_index("c")
    s = lax.axis_index("s")
    wid = s * NC + c
    idx_base = wid * IDXR
    lane = lax.iota(jnp.int32, 16)

    for p in range(NPASS):
        def zb(i, carry):
            for j in range(D // 16):
                rows_v[i, pl.ds(j * 16, 16)] = jnp.zeros((16,),
                                                         jnp.float32)
            return carry
        lax.fori_loop(0, ZSTRIPE + 32, zb, 0)

        pltpu.sync_copy(rows_v.at[pl.ds(0, ZSTRIPE)],
                        acc_s.at[pl.ds(s * ZSTRIPE, ZSTRIPE)])
        @pl.when(s == NS - 1)
        def _():
            ztail = NH - NS * ZSTRIPE
            pltpu.sync_copy(rows_v.at[pl.ds(0, ztail)],
                            acc_s.at[pl.ds(NS * ZSTRIPE, ztail)])
        plsc.subcore_barrier()

        def chunk_body(ci, carry):
            r0 = idx_base + ci * KR
            pltpu.sync_copy(src_hbm.at[pl.ds(r0, KR)], src_v)
            pltpu.sync_copy(dst_hbm.at[pl.ds(r0, KR)], dst_v)
            pltpu.sync_copy(ew16_hbm.at[pl.ds(wid * EPW + ci * K, K)],
                            w16_v)

            cps = [pltpu.async_copy(x_hbm.at[src_v.at[r]],
                                    rows_v.at[pl.ds(r * KC, KC)], sem)
                   for r in range(KR)]

            for r in range(KR):
                for off in (0, 16, 32, 34):
                    dd = dst_v[r, pl.ds(off, 16)]
                    ll = dd - (p * PN)
                    bad = (ll < 0) | (ll >= PN)
                    dstl_v[r, pl.ds(off, 16)] = jnp.where(bad, PN + lane,
                                                          ll)

            for cp in cps:
                cp.wait()

            def edge_body(e2, carry2):
                for u in range(2):
                    e = e2 * 2 + u
                    wv = w16_v[e, :]
                    for q in range(D // 16):
                        rr = rows_v[e, pl.ds(q * 16, 16)]
                        rows_v[e, pl.ds(q * 16, 16)] = rr * wv
                return carry2
            lax.fori_loop(0, K // 2, edge_body, 0)

            for r in range(KR):
                pltpu.sync_copy(rows_v.at[pl.ds(r * KC, KC)],
                                acc_s.at[dstl_v.at[r]], add=True)
            return carry
        lax.fori_loop(0, NCHUNK, chunk_body, 0)
        plsc.subcore_barrier()

        rows_p = PN if p < NPASS - 1 else N - (NPASS - 1) * PN
        obase = p * PN + s * ZSTRIPE
        tail = rows_p - NS * ZSTRIPE

        def writeout(s_out):
            pltpu.sync_copy(acc_s.at[pl.ds(s * ZSTRIPE, ZSTRIPE)],
                            s_out.at[pl.ds(obase, ZSTRIPE)])
            if tail > 0:
                @pl.when(s == NS - 1)
                def _():
                    pltpu.sync_copy(acc_s.at[pl.ds(NS * ZSTRIPE, tail)],
                                    s_out.at[pl.ds(p * PN + NS * ZSTRIPE,
                                                   tail)])

        @pl.when(c == 0)
        def _():
            writeout(s0_hbm)

        @pl.when(c == 1)
        def _():
            writeout(s1_hbm)

        plsc.subcore_barrier()


_sc_agg = pl.kernel(
    _sc_agg_body,
    out_type=[jax.ShapeDtypeStruct((N, D), jnp.float32),
              jax.ShapeDtypeStruct((N, D), jnp.float32)],
    mesh=plsc.VectorSubcoreMesh(core_axis_name="c", subcore_axis_name="s"),
    scratch_types=[
        pltpu.VMEM((KR, KC), jnp.int32),
        pltpu.VMEM((KR, KC), jnp.int32),
        pltpu.VMEM((KR, KC), jnp.int32),
        pltpu.VMEM((K, DEGW), jnp.float32),
        pltpu.VMEM((K, D), jnp.float32),
        pltpu.VMEM_SHARED((NH, D), jnp.float32),
        pltpu.SemaphoreType.DMA,
    ],
    name="sc_gin_agg",
)

BN = 2000


BQ = 8000


def _tc_bcast_body(w_ref, o_ref):
    o_ref[...] = jnp.broadcast_to(w_ref[...], (BQ, DEGW))


def _tc_bcast(ew_col):
    return pl.pallas_call(
        _tc_bcast_body,
        grid=(E // BQ,),
        in_specs=[pl.BlockSpec((BQ, 1), lambda i: (i, 0))],
        out_specs=pl.BlockSpec((BQ, DEGW), lambda i: (i, 0)),
        out_shape=jax.ShapeDtypeStruct((E, DEGW), jnp.float32),
    )(ew_col)


def _agg_block(s0, s1, deg):
    safe = jnp.where(deg > 0, deg, 1.0)
    return (s0 + s1) / safe


def _tc_layer_body(x_ref, s0_ref, s1_ref, deg_ref, w_ref, b_ref,
                   o_ref):
    z = x_ref[...] + _agg_block(s0_ref[...], s1_ref[...], deg_ref[...])
    h = lax.dot_general(z, w_ref[...], (((1,), (1,)), ((), ())),
                        preferred_element_type=jnp.float32) + b_ref[...]
    o_ref[...] = jnp.maximum(h, 0.0)


def _tc_layer(x, s0, s1, deg, w, b):
    return pl.pallas_call(
        _tc_layer_body,
        grid=(N // BN,),
        in_specs=[
            pl.BlockSpec((BN, D), lambda i: (i, 0)),
            pl.BlockSpec((BN, D), lambda i: (i, 0)),
            pl.BlockSpec((BN, D), lambda i: (i, 0)),
            pl.BlockSpec((BN, 1), lambda i: (i, 0)),
            pl.BlockSpec((D, D), lambda i: (0, 0)),
            pl.BlockSpec((1, D), lambda i: (0, 0)),
        ],
        out_specs=pl.BlockSpec((BN, D), lambda i: (i, 0)),
        out_shape=jax.ShapeDtypeStruct((N, D), jnp.float32),
    )(x, s0, s1, deg, w, b)


def _tc_head_body(h_ref, s0_ref, s1_ref, deg_ref, nw_ref,
                  w2_ref, b2_ref, wd_ref, bd_ref, wc_ref, bc_ref,
                  o_ref, acc_ref):
    i = pl.program_id(0)

    @pl.when(i == 0)
    def _():
        acc_ref[...] = jnp.zeros_like(acc_ref)

    z = h_ref[...] + _agg_block(s0_ref[...], s1_ref[...], deg_ref[...])
    h2 = lax.dot_general(z, w2_ref[...], (((1,), (1,)), ((), ())),
                         preferred_element_type=jnp.float32) + b2_ref[...]
    h2 = jnp.maximum(h2, 0.0)
    acc_ref[...] += lax.dot_general(nw_ref[...], h2,
                                    (((0,), (0,)), ((), ())),
                                    preferred_element_type=jnp.float32)

    @pl.when(i == pl.num_programs(0) - 1)
    def _():
        hg = acc_ref[...] * (1.0 / N)
        o1 = lax.dot_general(hg, wd_ref[...], (((1,), (1,)), ((), ())),
                             preferred_element_type=jnp.float32) + bd_ref[...]
        o1 = jnp.maximum(o1, 0.0)
        o_ref[...] = lax.dot_general(o1, wc_ref[...],
                                     (((1,), (1,)), ((), ())),
                                     preferred_element_type=jnp.float32) \
            + bc_ref[...]


def _tc_head(h, s0, s1, deg, nw_col, w2, b2, wd, bd, wc, bc):
    nh = wd.shape[0]
    nc = wc.shape[0]
    return pl.pallas_call(
        _tc_head_body,
        grid=(N // BN,),
        in_specs=[
            pl.BlockSpec((BN, D), lambda i: (i, 0)),
            pl.BlockSpec((BN, D), lambda i: (i, 0)),
            pl.BlockSpec((BN, D), lambda i: (i, 0)),
            pl.BlockSpec((BN, 1), lambda i: (i, 0)),
            pl.BlockSpec((BN, 1), lambda i: (i, 0)),
            pl.BlockSpec((D, D), lambda i: (0, 0)),
            pl.BlockSpec((1, D), lambda i: (0, 0)),
            pl.BlockSpec((nh, D), lambda i: (0, 0)),
            pl.BlockSpec((1, nh), lambda i: (0, 0)),
            pl.BlockSpec((nc, nh), lambda i: (0, 0)),
            pl.BlockSpec((1, nc), lambda i: (0, 0)),
        ],
        out_specs=pl.BlockSpec((1, nc), lambda i: (0, 0)),
        out_shape=jax.ShapeDtypeStruct((1, nc), jnp.float32),
        scratch_shapes=[pltpu.VMEM((1, D), jnp.float32)],
    )(h, s0, s1, deg, nw_col, w2, b2, wd, bd, wc, bc)


@jax.jit
def kernel(in_feat, edge_index, edge_weights, node_weights,
           W1, b1, W2, b2, Wd, bd, Wc, bc):
    src2d = edge_index[0].reshape(E // KC, KC)
    dst2d = edge_index[1].reshape(E // KC, KC)
    ew16 = _tc_bcast(edge_weights.reshape(E, 1))

    deg = jax.ops.segment_sum(edge_weights, edge_index[1],
                              num_segments=N).reshape(N, 1)
    s0, s1 = _sc_agg(in_feat, src2d, dst2d, ew16)
    h = _tc_layer(in_feat, s0, s1, deg, W1, b1.reshape(1, -1))
    t0, t1 = _sc_agg(h, src2d, dst2d, ew16)
    out = _tc_head(h, t0, t1, deg, node_weights.reshape(N, 1),
                   W2, b2.reshape(1, -1), Wd, bd.reshape(1, -1),
                   Wc, bc.reshape(1, -1))
    return out
